# baseline (speedup 1.0000x reference)
import jax
import jax.numpy as jnp
from jax.experimental import pallas as pl
from jax.experimental.pallas import tpu as pltpu

CHUNK = 1024
W_SIZES = (4096, 4096)


def kernel(inputs, table):
    batch, seq = inputs.shape
    n_rows, d_model = table.shape
    n_chunks = seq // CHUNK
    w_offs = []
    off = 0
    for ln in W_SIZES:
        w_offs.append(off)
        off += ln
    assert off == seq

    def body(table_hbm, out_hbm, vmem, rsem, wsem):
        def read_copy(c):
            return pltpu.make_async_copy(
                table_hbm.at[pl.ds(c * CHUNK, CHUNK), :],
                vmem.at[pl.ds(c * CHUNK, CHUNK), :],
                rsem.at[c],
            )

        def write_copy(k, b):
            o, ln = w_offs[k], W_SIZES[k]
            return pltpu.make_async_copy(
                vmem.at[pl.ds(o, ln), :],
                out_hbm.at[b, pl.ds(o, ln), :],
                wsem.at[k, b],
            )

        for c in range(n_chunks):
            read_copy(c).start()

        chunks_waited = 0
        for k in range(len(W_SIZES)):
            need = (w_offs[k] + W_SIZES[k]) // CHUNK
            for c in range(chunks_waited, need):
                read_copy(c).wait()
            chunks_waited = need
            for b in range(batch):
                write_copy(k, b).start()

        for k in range(len(W_SIZES)):
            for b in range(batch):
                write_copy(k, b).wait()

    return pl.pallas_call(
        body,
        in_specs=[pl.BlockSpec(memory_space=pl.ANY)],
        out_specs=pl.BlockSpec(memory_space=pl.ANY),
        out_shape=jax.ShapeDtypeStruct((batch, seq, d_model), table.dtype),
        scratch_shapes=[
            pltpu.VMEM((seq, d_model), table.dtype),
            pltpu.SemaphoreType.DMA((n_chunks,)),
            pltpu.SemaphoreType.DMA((len(W_SIZES), batch)),
        ],
    )(table)
